# baseline (speedup 1.0000x reference)
import functools

import jax
import jax.numpy as jnp
import numpy as np
from jax.experimental import pallas as pl
from jax.experimental.pallas import tpu as pltpu

_SCALES = (
    (1, 480, 476, 238, 234, 117, 26, 9),
    (2, 240, 236, 118, 114, 57, 19, 6),
    (4, 120, 116, 58, 54, 27, 18, 3),
)

_B = 2048
_BB = 128
_NB = _B // _BB
_F = 6432
_FP = 6528
_BM, _BK, _BN = 544, 2176, 1024


def _mk_band(w2, TL):
    cols = TL + 4
    m4 = jnp.zeros((TL, 32, cols, 16), jnp.float32)
    for k in range(5):
        e = jnp.asarray(np.eye(TL, cols, k, dtype=np.float32))
        m4 = m4 + e[:, None, :, None] * w2[:, :, k][None, :, None, :]
    return m4.reshape(TL * 32, cols * 16)


def _feat_kernel(x_ref, w1b_ref, b1b_ref, b2b_ref, m1_ref, m2_ref, m4_ref,
                 o_ref):
    xb = x_ref[...]
    w1b = w1b_ref[...]
    b1b = b1b_ref[...]
    b2b = b2b_ref[...]
    mrefs = {1: m1_ref, 2: m2_ref, 4: m4_ref}
    outs = []
    for (s, L0, L1, L1p, L2, L2p, TL, NT) in _SCALES:
        if s == 1:
            cg = xb
        else:
            cg = jnp.mean(xb.reshape(L0, s, 4, _BB), axis=1)
        acc = jnp.broadcast_to(b1b[None], (L1, 16, _BB))
        for c in range(4):
            xc = cg[:, c:c + 1, :]
            for k in range(5):
                acc = acc + w1b[c * 5 + k][None] * xc[k:k + L1]
        y1p = jnp.max(jnp.maximum(acc, 0.0).reshape(L1p, 2, 16, _BB), axis=1)
        flat = y1p.reshape(L1p * 16, _BB)
        m = mrefs[s][...]
        tiles = []
        for t in range(NT):
            seg = flat[t * TL * 16: (t * TL + TL + 4) * 16, :]
            yt = jnp.dot(m, seg, preferred_element_type=jnp.float32)
            yt = jnp.maximum(yt.reshape(TL, 32, _BB) + b2b[None], 0.0)
            tiles.append(yt)
        y2 = jnp.concatenate(tiles, axis=0)
        outs.append(jnp.max(y2.reshape(L2p, 2, 32, _BB), axis=1))
    o_ref[...] = jnp.concatenate(outs, axis=0)


def _dense_kernel(nk, relu, w_ref, h_ref, b_ref, o_ref):
    k = pl.program_id(2)
    acc = jnp.dot(w_ref[...], h_ref[...], preferred_element_type=jnp.float32)

    @pl.when(k == 0)
    def _():
        o_ref[...] = acc + jnp.concatenate([b_ref[...]] * (_BN // 128), axis=1)

    @pl.when(k > 0)
    def _():
        o_ref[...] = o_ref[...] + acc

    if relu:
        @pl.when(k == nk - 1)
        def _():
            o_ref[...] = jnp.maximum(o_ref[...], 0.0)


def _head_kernel(nk, w_ref, h_ref, b_ref, o_ref):
    k = pl.program_id(0)
    acc = jnp.dot(w_ref[...], h_ref[...], preferred_element_type=jnp.float32)

    @pl.when(k == 0)
    def _():
        o_ref[...] = acc + jnp.concatenate([b_ref[...]] * (_B // 128), axis=1)

    @pl.when(k > 0)
    def _():
        o_ref[...] = o_ref[...] + acc


def _features(xt, w1b, b1b, b2b, m1, m2, m4):
    return pl.pallas_call(
        _feat_kernel,
        grid=(_NB,),
        in_specs=[
            pl.BlockSpec((480, 4, _BB), lambda i: (0, 0, i)),
            pl.BlockSpec((20, 16, 128), lambda i: (0, 0, 0)),
            pl.BlockSpec((16, 128), lambda i: (0, 0)),
            pl.BlockSpec((32, 128), lambda i: (0, 0)),
            pl.BlockSpec(m1.shape, lambda i: (0, 0)),
            pl.BlockSpec(m2.shape, lambda i: (0, 0)),
            pl.BlockSpec(m4.shape, lambda i: (0, 0)),
        ],
        out_specs=pl.BlockSpec((201, 32, _BB), lambda i: (0, 0, i)),
        out_shape=jax.ShapeDtypeStruct((201, 32, _B), jnp.float32),
        compiler_params=pltpu.CompilerParams(
            dimension_semantics=("parallel",)),
    )(xt, w1b, b1b, b2b, m1, m2, m4)


def _dense(wlp, h, blb, relu):
    nn, nm, nk = _B // _BN, _FP // _BM, _FP // _BK
    return pl.pallas_call(
        functools.partial(_dense_kernel, nk, relu),
        grid=(nn, nm, nk),
        in_specs=[
            pl.BlockSpec((_BM, _BK), lambda n, m, k: (m, k)),
            pl.BlockSpec((_BK, _BN), lambda n, m, k: (k, n)),
            pl.BlockSpec((_BM, 128), lambda n, m, k: (m, 0)),
        ],
        out_specs=pl.BlockSpec((_BM, _BN), lambda n, m, k: (m, n)),
        out_shape=jax.ShapeDtypeStruct((_FP, _B), jnp.float32),
        compiler_params=pltpu.CompilerParams(
            dimension_semantics=("parallel", "parallel", "arbitrary")),
    )(wlp, h, blb)


def _head(wop, h, bob):
    nk = _FP // _BK
    return pl.pallas_call(
        functools.partial(_head_kernel, nk),
        grid=(nk,),
        in_specs=[
            pl.BlockSpec((8, _BK), lambda k: (0, k)),
            pl.BlockSpec((_BK, _B), lambda k: (k, 0)),
            pl.BlockSpec((8, 128), lambda k: (0, 0)),
        ],
        out_specs=pl.BlockSpec((8, _B), lambda k: (0, 0)),
        out_shape=jax.ShapeDtypeStruct((8, _B), jnp.float32),
        compiler_params=pltpu.CompilerParams(
            dimension_semantics=("arbitrary",)),
    )(wop, h, bob)


@jax.jit
def _impl(x, w1, b1, w2, b2, wl, bl, wo, bo):
    xt = jnp.transpose(x, (2, 1, 0))
    w1b = jnp.broadcast_to(
        w1.transpose(1, 2, 0).reshape(20, 16)[:, :, None], (20, 16, _BB))
    b1b = jnp.broadcast_to(b1[:, None], (16, _BB))
    b2b = jnp.broadcast_to(b2[:, None], (32, _BB))
    m1, m2, m4 = _mk_band(w2, 26), _mk_band(w2, 19), _mk_band(w2, 18)
    ft = _features(xt, w1b, b1b, b2b, m1, m2, m4)
    h = ft.transpose(1, 0, 2).reshape(_F, _B)
    h = jnp.pad(h, ((0, _FP - _F), (0, 0)))
    wlp = jnp.pad(wl, ((0, _FP - _F), (0, _FP - _F)))
    blb = jnp.broadcast_to(jnp.pad(bl, (0, _FP - _F))[:, None], (_FP, 128))
    for _ in range(4):
        h = _dense(wlp, h, blb, True)
    wop = jnp.pad(wo, ((0, 3), (0, _FP - _F)))
    bob = jnp.broadcast_to(jnp.pad(bo, (0, 3))[:, None], (8, 128))
    out = _head(wop, h, bob)
    return out[:5].T


def kernel(x, w1, b1, w2, b2, wl, bl, wo, bo):
    return _impl(x, w1, b1, w2, b2, wl, bl, wo, bo)


# speedup vs baseline: 1.5128x; 1.5128x over previous
import jax
import jax.numpy as jnp
import numpy as np
from jax.experimental import pallas as pl
from jax.experimental.pallas import tpu as pltpu

_SCALES = (
    (1, 480, 476, 238, 234, 117, 26, 9),
    (2, 240, 236, 118, 114, 57, 19, 6),
    (4, 120, 116, 58, 54, 27, 18, 3),
)

_B = 2048
_BB = 128
_NB = _B // _BB
_F = 6432
_BM, _BN = 536, 1024


def _mk_band(w2, TL):
    cols = TL + 4
    m4 = jnp.zeros((TL, 32, cols, 16), jnp.float32)
    for k in range(5):
        e = jnp.asarray(np.eye(TL, cols, k, dtype=np.float32))
        m4 = m4 + e[:, None, :, None] * w2[:, :, k][None, :, None, :]
    return m4.reshape(TL * 32, cols * 16)


def _feat_kernel(x_ref, w1b_ref, b1b_ref, b2b_ref, m1_ref, m2_ref, m4_ref,
                 o_ref):
    xb = x_ref[...]
    w1b = w1b_ref[...]
    b1b = b1b_ref[...]
    b2b = b2b_ref[...]
    mrefs = {1: m1_ref, 2: m2_ref, 4: m4_ref}
    outs = []
    for (s, L0, L1, L1p, L2, L2p, TL, NT) in _SCALES:
        if s == 1:
            cg = xb
        else:
            cg = jnp.mean(xb.reshape(L0, s, 4, _BB), axis=1)
        acc = jnp.broadcast_to(b1b[None], (L1, 16, _BB))
        for c in range(4):
            xc = cg[:, c:c + 1, :]
            for k in range(5):
                acc = acc + w1b[c * 5 + k][None] * xc[k:k + L1]
        y1p = jnp.max(jnp.maximum(acc, 0.0).reshape(L1p, 2, 16, _BB), axis=1)
        flat = y1p.reshape(L1p * 16, _BB)
        m = mrefs[s][...]
        tiles = []
        for t in range(NT):
            seg = flat[t * TL * 16: (t * TL + TL + 4) * 16, :]
            yt = jnp.dot(m, seg, preferred_element_type=jnp.float32)
            yt = jnp.maximum(yt.reshape(TL, 32, _BB) + b2b[None], 0.0)
            tiles.append(yt)
        y2 = jnp.concatenate(tiles, axis=0)
        outs.append(jnp.max(y2.reshape(L2p, 2, 32, _BB), axis=1))
    o_ref[...] = jnp.concatenate(outs, axis=0).astype(jnp.bfloat16)


def _dense_kernel(w_ref, h_ref, b_ref, o_ref):
    acc = jnp.dot(w_ref[...], h_ref[...], preferred_element_type=jnp.float32)
    bb = jnp.concatenate([b_ref[...]] * (_BN // 128), axis=1)
    o_ref[...] = jnp.maximum(acc + bb, 0.0).astype(jnp.bfloat16)


def _head_kernel(w_ref, h_ref, b_ref, o_ref):
    acc = jnp.dot(w_ref[...], h_ref[...], preferred_element_type=jnp.float32)
    o_ref[...] = acc + jnp.concatenate([b_ref[...]] * (_B // 128), axis=1)


def _features(xt, w1b, b1b, b2b, m1, m2, m4):
    return pl.pallas_call(
        _feat_kernel,
        grid=(_NB,),
        in_specs=[
            pl.BlockSpec((480, 4, _BB), lambda i: (0, 0, i)),
            pl.BlockSpec((20, 16, 128), lambda i: (0, 0, 0)),
            pl.BlockSpec((16, 128), lambda i: (0, 0)),
            pl.BlockSpec((32, 128), lambda i: (0, 0)),
            pl.BlockSpec(m1.shape, lambda i: (0, 0)),
            pl.BlockSpec(m2.shape, lambda i: (0, 0)),
            pl.BlockSpec(m4.shape, lambda i: (0, 0)),
        ],
        out_specs=pl.BlockSpec((201, 32, _BB), lambda i: (0, 0, i)),
        out_shape=jax.ShapeDtypeStruct((201, 32, _B), jnp.bfloat16),
        compiler_params=pltpu.CompilerParams(
            dimension_semantics=("parallel",)),
    )(xt, w1b, b1b, b2b, m1, m2, m4)


def _dense(wlb, h, blb):
    nn, nm = _B // _BN, _F // _BM
    return pl.pallas_call(
        _dense_kernel,
        grid=(nn, nm),
        in_specs=[
            pl.BlockSpec((_BM, _F), lambda n, m: (m, 0)),
            pl.BlockSpec((_F, _BN), lambda n, m: (0, n)),
            pl.BlockSpec((_BM, 128), lambda n, m: (m, 0)),
        ],
        out_specs=pl.BlockSpec((_BM, _BN), lambda n, m: (m, n)),
        out_shape=jax.ShapeDtypeStruct((_F, _B), jnp.bfloat16),
        compiler_params=pltpu.CompilerParams(
            dimension_semantics=("parallel", "parallel")),
    )(wlb, h, blb)


def _head(wob, h, bob):
    return pl.pallas_call(
        _head_kernel,
        grid=(1,),
        in_specs=[
            pl.BlockSpec((8, _F), lambda k: (0, 0)),
            pl.BlockSpec((_F, _B), lambda k: (0, 0)),
            pl.BlockSpec((8, 128), lambda k: (0, 0)),
        ],
        out_specs=pl.BlockSpec((8, _B), lambda k: (0, 0)),
        out_shape=jax.ShapeDtypeStruct((8, _B), jnp.float32),
        compiler_params=pltpu.CompilerParams(
            dimension_semantics=("arbitrary",)),
    )(wob, h, bob)


@jax.jit
def _impl(x, w1, b1, w2, b2, wl, bl, wo, bo):
    xt = jnp.transpose(x, (2, 1, 0))
    w1b = jnp.broadcast_to(
        w1.transpose(1, 2, 0).reshape(20, 16)[:, :, None], (20, 16, _BB))
    b1b = jnp.broadcast_to(b1[:, None], (16, _BB))
    b2b = jnp.broadcast_to(b2[:, None], (32, _BB))
    m1, m2, m4 = _mk_band(w2, 26), _mk_band(w2, 19), _mk_band(w2, 18)
    ft = _features(xt, w1b, b1b, b2b, m1, m2, m4)
    h = ft.transpose(1, 0, 2).reshape(_F, _B)
    wlb = wl.astype(jnp.bfloat16)
    blb = jnp.broadcast_to(bl[:, None], (_F, 128))
    for _ in range(4):
        h = _dense(wlb, h, blb)
    wob = jnp.pad(wo, ((0, 3), (0, 0))).astype(jnp.bfloat16)
    bob = jnp.broadcast_to(jnp.pad(bo, (0, 3))[:, None], (8, 128))
    out = _head(wob, h, bob)
    return out[:5].T


def kernel(x, w1, b1, w2, b2, wl, bl, wo, bo):
    return _impl(x, w1, b1, w2, b2, wl, bl, wo, bo)
